# Initial kernel scaffold; baseline (speedup 1.0000x reference)
import jax
import jax.numpy as jnp
from jax.experimental import pallas as pl


def kernel(J, h, m0, i_seq, r_seq, steps):
    raise NotImplementedError("write your pallas kernel here")


# speedup vs baseline: 16.0895x; 16.0895x over previous
import functools

import jax
import jax.numpy as jnp
from jax.experimental import pallas as pl
from jax.experimental.pallas import tpu as pltpu

N = 8192


def _pbit_step(i_ref, r_ref, h_ref, row_ref, m0_ref, out_ref, m_ref):
    t = pl.program_id(0)
    steps = pl.num_programs(0)

    @pl.when(t == 0)
    def _():
        m_ref[...] = m0_ref[...]

    i = i_ref[t]
    m = m_ref[...]
    I_i = jnp.sum(row_ref[0] * m) + h_ref[i]
    p = jnp.tanh(I_i)
    val = jnp.where(p >= r_ref[t], jnp.float32(1.0), jnp.float32(-1.0))
    lane = jax.lax.broadcasted_iota(jnp.int32, (1, N), 1)
    m_ref[...] = jnp.where(lane == i, val, m)

    @pl.when(t == steps - 1)
    def _():
        out_ref[...] = m_ref[...]


def kernel(J, h, m0, i_seq, r_seq, steps):
    del steps
    T = i_seq.shape[0]
    grid_spec = pltpu.PrefetchScalarGridSpec(
        num_scalar_prefetch=3,
        grid=(T,),
        in_specs=[
            pl.BlockSpec((1, 1, N), lambda t, i_ref, r_ref, h_ref: (i_ref[t], 0, 0)),
            pl.BlockSpec((1, N), lambda t, i_ref, r_ref, h_ref: (0, 0)),
        ],
        out_specs=pl.BlockSpec((1, N), lambda t, i_ref, r_ref, h_ref: (0, 0)),
        scratch_shapes=[pltpu.VMEM((1, N), jnp.float32)],
    )
    out = pl.pallas_call(
        _pbit_step,
        grid_spec=grid_spec,
        out_shape=jax.ShapeDtypeStruct((1, N), jnp.float32),
    )(i_seq, r_seq, h, J.reshape(N, 1, N), m0.reshape(1, N))
    return out.reshape(N)


# speedup vs baseline: 33.8936x; 2.1066x over previous
import functools

import jax
import jax.numpy as jnp
from jax import lax
from jax.experimental import pallas as pl
from jax.experimental.pallas import tpu as pltpu
from jax.experimental.pallas import tpu_sc as plsc

N = 8192
T = 4096
K = 512
NCHUNK = T // K
RA = 128

NWORK = 32
RPW = T // NWORK
BR = 8
NBATCH = RPW // BR


def _matvec_body(j_ref, m0_ref, h_ref, out_ref):
    prod = j_ref[...] * m0_ref[...]
    out_ref[...] = jnp.sum(prod, axis=1, keepdims=True) + h_ref[...]


def _matvec(J, m0, h):
    return pl.pallas_call(
        _matvec_body,
        grid=(N // RA,),
        in_specs=[
            pl.BlockSpec((RA, N), lambda b: (b, 0)),
            pl.BlockSpec((1, N), lambda b: (0, 0)),
            pl.BlockSpec((RA, 1), lambda b: (b, 0)),
        ],
        out_specs=pl.BlockSpec((RA, 1), lambda b: (b, 0)),
        out_shape=jax.ShapeDtypeStruct((N, 1), jnp.float32),
    )(J, m0.reshape(1, N), h.reshape(N, 1))


def _sc_extract_body(jflat_hbm, idx_hbm, i0_hbm, s_hbm, base_hbm,
                     idx_all_v, i0_v, row_a, row_b, st_a, st_b, bstage_v,
                     idx_my_s, dma_a, dma_b, w_a, w_b):
    wid = lax.axis_index("s") * 2 + lax.axis_index("c")
    row0 = wid * RPW

    pltpu.sync_copy(idx_hbm, idx_all_v)
    pltpu.sync_copy(i0_hbm, i0_v)
    def smem_fill(k, carry):
        vec = idx_all_v[pl.ds(row0 + k * 16, 16)]
        for l in range(16):
            idx_my_s[k * 16 + l] = vec[l]
        return carry
    lax.fori_loop(0, RPW // 16, smem_fill, 0)

    def base_body(k, carry):
        idxs = idx_all_v[pl.ds(row0 + k * 16, 16)]
        bstage_v[pl.ds(k * 16, 16)] = plsc.load_gather(i0_v, [idxs])
        return carry
    lax.fori_loop(0, RPW // 16, base_body, 0)
    pltpu.sync_copy(bstage_v, base_hbm.at[pl.ds(row0, RPW)])

    def fire(r, buf, sem):
        i = idx_my_s[r]
        pltpu.async_copy(jflat_hbm.at[pl.ds(i * N, N)], buf, sem)

    def wait_row(buf, sem):
        pltpu.make_async_copy(jflat_hbm.at[pl.ds(0, N)], buf, sem).wait()

    def extract(buf, stage):
        def col_body(k, carry):
            idxs = idx_all_v[pl.ds(k * 16, 16)]
            stage[pl.ds(k * 16, 16)] = plsc.load_gather(buf, [idxs])
            return carry
        lax.fori_loop(0, T // 16, col_body, 0)

    fire(0, row_a, dma_a)

    def pair_body(j, carry):
        t0 = row0 + 2 * j
        t1 = t0 + 1
        rn = jnp.minimum(2 * j + 2, RPW - 1)
        wait_row(row_a, dma_a)
        fire(2 * j + 1, row_b, dma_b)

        @pl.when(j > 0)
        def _():
            pltpu.make_async_copy(st_a, s_hbm.at[t0], w_a).wait()
        extract(row_a, st_a)
        pltpu.async_copy(st_a, s_hbm.at[t0], w_a)

        wait_row(row_b, dma_b)
        fire(rn, row_a, dma_a)

        @pl.when(j > 0)
        def _():
            pltpu.make_async_copy(st_b, s_hbm.at[t1], w_b).wait()
        extract(row_b, st_b)
        pltpu.async_copy(st_b, s_hbm.at[t1], w_b)
        return carry

    lax.fori_loop(0, RPW // 2, pair_body, 0)
    wait_row(row_a, dma_a)
    pltpu.make_async_copy(st_a, s_hbm.at[0], w_a).wait()
    pltpu.make_async_copy(st_b, s_hbm.at[0], w_b).wait()


def _sc_extract(J, i_seq, I0):
    mesh = plsc.VectorSubcoreMesh(core_axis_name="c", subcore_axis_name="s")
    f = pl.kernel(
        _sc_extract_body,
        out_type=(
            jax.ShapeDtypeStruct((T, T), jnp.float32),
            jax.ShapeDtypeStruct((T,), jnp.float32),
        ),
        mesh=mesh,
        compiler_params=pltpu.CompilerParams(needs_layout_passes=False),
        scratch_types=[
            pltpu.VMEM((T,), jnp.int32),
            pltpu.VMEM((N,), jnp.float32),
            pltpu.VMEM((N,), jnp.float32),
            pltpu.VMEM((N,), jnp.float32),
            pltpu.VMEM((T,), jnp.float32),
            pltpu.VMEM((T,), jnp.float32),
            pltpu.VMEM((RPW,), jnp.float32),
            pltpu.SMEM((RPW,), jnp.int32),
            pltpu.SemaphoreType.DMA,
            pltpu.SemaphoreType.DMA,
            pltpu.SemaphoreType.DMA,
            pltpu.SemaphoreType.DMA,
        ],
    )
    return f(J.reshape(N * N), i_seq, I0.reshape(N))


def _chunk_body(i_sp, r_sp, srow_ref, sdiag_ref, base_ref, m0_ref,
                mout_ref, field_ref, m_smem, sem):
    c = pl.program_id(0)

    @pl.when(c == 0)
    def _():
        field_ref[...] = base_ref[...]
        pltpu.make_async_copy(m0_ref, m_smem, sem).start()
        pltpu.make_async_copy(m0_ref, m_smem, sem).wait()

    lanesK = lax.broadcasted_iota(jnp.int32, (1, K), 1)
    corr0 = field_ref[c, :].reshape(1, K)

    def step(tl, carry):
        deltas, corr = carry
        tg = c * K + tl
        i = i_sp[tg]
        r = r_sp[tg]
        I = jnp.sum(jnp.where(lanesK == tl, corr, 0.0))
        p = jnp.tanh(I)
        v = jnp.where(p >= r, jnp.float32(1.0), jnp.float32(-1.0))
        mprev = m_smem[0, i]
        d = v - mprev
        m_smem[0, i] = v
        srow = sdiag_ref[tl, :].reshape(1, K)
        corr = corr + srow * d
        deltas = jnp.where(lanesK == tl, d, deltas)
        return deltas, corr

    z = jnp.zeros((1, K), jnp.float32)
    deltas, _ = lax.fori_loop(0, K, step, (z, corr0))

    upd = jnp.dot(deltas, srow_ref[...], precision=lax.Precision.HIGHEST,
                  preferred_element_type=jnp.float32)
    field_ref[...] = field_ref[...] + upd.reshape(NCHUNK, K)

    @pl.when(c == pl.num_programs(0) - 1)
    def _():
        pltpu.make_async_copy(m_smem, mout_ref, sem).start()
        pltpu.make_async_copy(m_smem, mout_ref, sem).wait()


def _chunk_scan(i_seq, r_seq, S, base, m0):
    grid_spec = pltpu.PrefetchScalarGridSpec(
        num_scalar_prefetch=2,
        grid=(NCHUNK,),
        in_specs=[
            pl.BlockSpec((K, T), lambda c, isp, rsp: (c, 0)),
            pl.BlockSpec((K, K), lambda c, isp, rsp: (c, c)),
            pl.BlockSpec((NCHUNK, K), lambda c, isp, rsp: (0, 0)),
            pl.BlockSpec((1, N), lambda c, isp, rsp: (0, 0)),
        ],
        out_specs=pl.BlockSpec((1, N), lambda c, isp, rsp: (0, 0)),
        scratch_shapes=[
            pltpu.VMEM((NCHUNK, K), jnp.float32),
            pltpu.SMEM((1, N), jnp.float32),
            pltpu.SemaphoreType.DMA,
        ],
    )
    return pl.pallas_call(
        _chunk_body,
        grid_spec=grid_spec,
        out_shape=jax.ShapeDtypeStruct((1, N), jnp.float32),
    )(i_seq, r_seq, S, S, base.reshape(NCHUNK, K), m0.reshape(1, N))


def kernel(J, h, m0, i_seq, r_seq, steps):
    del steps
    I0 = _matvec(J, m0, h)
    S, base = _sc_extract(J, i_seq, I0)
    m = _chunk_scan(i_seq, r_seq, S, base, m0)
    return m.reshape(N)


# speedup vs baseline: 36.4879x; 1.0765x over previous
import functools

import jax
import jax.numpy as jnp
from jax import lax
from jax.experimental import pallas as pl
from jax.experimental.pallas import tpu as pltpu
from jax.experimental.pallas import tpu_sc as plsc

N = 8192
T = 4096
K = 512
NCHUNK = T // K
RA = 128

NWORK = 32
RPW = T // NWORK
BR = 8
NBATCH = RPW // BR


def _matvec_body(j_ref, m0_ref, h_ref, out_ref):
    prod = j_ref[...] * m0_ref[...]
    out_ref[...] = jnp.sum(prod, axis=1, keepdims=True) + h_ref[...]


def _matvec(J, m0, h):
    return pl.pallas_call(
        _matvec_body,
        grid=(N // RA,),
        in_specs=[
            pl.BlockSpec((RA, N), lambda b: (b, 0)),
            pl.BlockSpec((1, N), lambda b: (0, 0)),
            pl.BlockSpec((RA, 1), lambda b: (b, 0)),
        ],
        out_specs=pl.BlockSpec((RA, 1), lambda b: (b, 0)),
        out_shape=jax.ShapeDtypeStruct((N, 1), jnp.float32),
    )(J, m0.reshape(1, N), h.reshape(N, 1))


def _sc_extract_body(j_hbm, idx_hbm, i0_hbm, s_hbm, base_hbm,
                     idx_all_v, i0_v, row_a, row_b, st_a, st_b, bstage_v,
                     idx_my_s, dma_a, dma_b, w_a, w_b):
    wid = lax.axis_index("s") * 2 + lax.axis_index("c")
    row0 = wid * RPW

    pltpu.sync_copy(idx_hbm, idx_all_v)
    pltpu.sync_copy(i0_hbm, i0_v)
    def smem_fill(k, carry):
        vec = idx_all_v[pl.ds(row0 + k * 16, 16)]
        for l in range(16):
            idx_my_s[k * 16 + l] = vec[l]
        return carry
    lax.fori_loop(0, RPW // 16, smem_fill, 0)

    def base_body(k, carry):
        idxs = idx_all_v[pl.ds(row0 + k * 16, 16)]
        bstage_v[pl.ds(k * 16, 16)] = plsc.load_gather(i0_v, [idxs])
        return carry
    lax.fori_loop(0, RPW // 16, base_body, 0)
    pltpu.sync_copy(bstage_v, base_hbm.at[pl.ds(row0, RPW)])

    def fire(r, buf, sem):
        i = idx_my_s[r]
        pltpu.async_copy(j_hbm.at[i], buf, sem)

    def wait_row(buf, sem):
        pltpu.make_async_copy(j_hbm.at[0], buf, sem).wait()

    def extract(buf, stage):
        def col_body(k, carry):
            idxs = idx_all_v[pl.ds(k * 16, 16)]
            stage[pl.ds(k * 16, 16)] = plsc.load_gather(buf, [idxs])
            return carry
        lax.fori_loop(0, T // 16, col_body, 0, unroll=8)

    fire(0, row_a, dma_a)

    def pair_body(j, carry):
        t0 = row0 + 2 * j
        t1 = t0 + 1
        rn = jnp.minimum(2 * j + 2, RPW - 1)
        wait_row(row_a, dma_a)
        fire(2 * j + 1, row_b, dma_b)

        @pl.when(j > 0)
        def _():
            pltpu.make_async_copy(st_a, s_hbm.at[t0], w_a).wait()
        extract(row_a, st_a)
        pltpu.async_copy(st_a, s_hbm.at[t0], w_a)

        wait_row(row_b, dma_b)
        fire(rn, row_a, dma_a)

        @pl.when(j > 0)
        def _():
            pltpu.make_async_copy(st_b, s_hbm.at[t1], w_b).wait()
        extract(row_b, st_b)
        pltpu.async_copy(st_b, s_hbm.at[t1], w_b)
        return carry

    lax.fori_loop(0, RPW // 2, pair_body, 0)
    wait_row(row_a, dma_a)
    pltpu.make_async_copy(st_a, s_hbm.at[0], w_a).wait()
    pltpu.make_async_copy(st_b, s_hbm.at[0], w_b).wait()


def _sc_extract(J, i_seq, I0):
    mesh = plsc.VectorSubcoreMesh(core_axis_name="c", subcore_axis_name="s")
    f = pl.kernel(
        _sc_extract_body,
        out_type=(
            jax.ShapeDtypeStruct((T, T), jnp.float32),
            jax.ShapeDtypeStruct((T,), jnp.float32),
        ),
        mesh=mesh,
        compiler_params=pltpu.CompilerParams(needs_layout_passes=False),
        scratch_types=[
            pltpu.VMEM((T,), jnp.int32),
            pltpu.VMEM((N,), jnp.float32),
            pltpu.VMEM((N,), jnp.float32),
            pltpu.VMEM((N,), jnp.float32),
            pltpu.VMEM((T,), jnp.float32),
            pltpu.VMEM((T,), jnp.float32),
            pltpu.VMEM((RPW,), jnp.float32),
            pltpu.SMEM((RPW,), jnp.int32),
            pltpu.SemaphoreType.DMA,
            pltpu.SemaphoreType.DMA,
            pltpu.SemaphoreType.DMA,
            pltpu.SemaphoreType.DMA,
        ],
    )
    return f(J, i_seq, I0.reshape(N))


def _chunk_body(i_sp, r_sp, srow_ref, sdiag_ref, base_ref, m0_ref,
                mout_ref, field_ref, m_smem, sem):
    c = pl.program_id(0)

    @pl.when(c == 0)
    def _():
        field_ref[...] = base_ref[...]
        pltpu.make_async_copy(m0_ref, m_smem, sem).start()
        pltpu.make_async_copy(m0_ref, m_smem, sem).wait()

    lanesK = lax.broadcasted_iota(jnp.int32, (1, K), 1)
    corr0 = field_ref[c, :].reshape(1, K)

    def step(tl, carry):
        deltas, corr = carry
        tg = c * K + tl
        i = i_sp[tg]
        r = r_sp[tg]
        I = jnp.sum(jnp.where(lanesK == tl, corr, 0.0))
        p = jnp.tanh(I)
        v = jnp.where(p >= r, jnp.float32(1.0), jnp.float32(-1.0))
        mprev = m_smem[0, i]
        d = v - mprev
        m_smem[0, i] = v
        srow = sdiag_ref[tl, :].reshape(1, K)
        corr = corr + srow * d
        deltas = jnp.where(lanesK == tl, d, deltas)
        return deltas, corr

    z = jnp.zeros((1, K), jnp.float32)
    deltas, _ = lax.fori_loop(0, K, step, (z, corr0))

    upd = jnp.dot(deltas, srow_ref[...], precision=lax.Precision.HIGHEST,
                  preferred_element_type=jnp.float32)
    field_ref[...] = field_ref[...] + upd.reshape(NCHUNK, K)

    @pl.when(c == pl.num_programs(0) - 1)
    def _():
        pltpu.make_async_copy(m_smem, mout_ref, sem).start()
        pltpu.make_async_copy(m_smem, mout_ref, sem).wait()


def _chunk_scan(i_seq, r_seq, S, base, m0):
    grid_spec = pltpu.PrefetchScalarGridSpec(
        num_scalar_prefetch=2,
        grid=(NCHUNK,),
        in_specs=[
            pl.BlockSpec((K, T), lambda c, isp, rsp: (c, 0)),
            pl.BlockSpec((K, K), lambda c, isp, rsp: (c, c)),
            pl.BlockSpec((NCHUNK, K), lambda c, isp, rsp: (0, 0)),
            pl.BlockSpec((1, N), lambda c, isp, rsp: (0, 0)),
        ],
        out_specs=pl.BlockSpec((1, N), lambda c, isp, rsp: (0, 0)),
        scratch_shapes=[
            pltpu.VMEM((NCHUNK, K), jnp.float32),
            pltpu.SMEM((1, N), jnp.float32),
            pltpu.SemaphoreType.DMA,
        ],
    )
    return pl.pallas_call(
        _chunk_body,
        grid_spec=grid_spec,
        out_shape=jax.ShapeDtypeStruct((1, N), jnp.float32),
    )(i_seq, r_seq, S, S, base.reshape(NCHUNK, K), m0.reshape(1, N))


def kernel(J, h, m0, i_seq, r_seq, steps):
    del steps
    I0 = _matvec(J, m0, h)
    S, base = _sc_extract(J, i_seq, I0)
    m = _chunk_scan(i_seq, r_seq, S, base, m0)
    return m.reshape(N)


# speedup vs baseline: 38.7617x; 1.0623x over previous
import functools

import jax
import jax.numpy as jnp
from jax import lax
from jax.experimental import pallas as pl
from jax.experimental.pallas import tpu as pltpu
from jax.experimental.pallas import tpu_sc as plsc

N = 8192
T = 4096
K = 512
NCHUNK = T // K
RA = 128

NWORK = 32
RPW = T // NWORK
NBUF = 4


def _matvec_body(j_ref, m0_ref, h_ref, out_ref):
    prod = j_ref[...] * m0_ref[...]
    out_ref[...] = jnp.sum(prod, axis=1, keepdims=True) + h_ref[...]


def _matvec(J, m0, h):
    return pl.pallas_call(
        _matvec_body,
        grid=(N // RA,),
        in_specs=[
            pl.BlockSpec((RA, N), lambda b: (b, 0)),
            pl.BlockSpec((1, N), lambda b: (0, 0)),
            pl.BlockSpec((RA, 1), lambda b: (b, 0)),
        ],
        out_specs=pl.BlockSpec((RA, 1), lambda b: (b, 0)),
        out_shape=jax.ShapeDtypeStruct((N, 1), jnp.float32),
    )(J, m0.reshape(1, N), h.reshape(N, 1))


def _sc_extract_body(j_hbm, idx_hbm, m0_hbm, s_hbm, m0g_hbm,
                     idx_all_v, m0_v, rb0, rb1, rb2, rb3,
                     st0, st1, st2, st3, gstage_v, idx_my_s,
                     d0, d1, d2, d3, w0, w1, w2, w3):
    wid = lax.axis_index("s") * 2 + lax.axis_index("c")
    row0 = wid * RPW
    rbufs = (rb0, rb1, rb2, rb3)
    stages = (st0, st1, st2, st3)
    dsems = (d0, d1, d2, d3)
    wsems = (w0, w1, w2, w3)

    pltpu.sync_copy(idx_hbm, idx_all_v)
    pltpu.sync_copy(m0_hbm, m0_v)

    def smem_fill(k, carry):
        vec = idx_all_v[pl.ds(row0 + k * 16, 16)]
        for l in range(16):
            idx_my_s[k * 16 + l] = vec[l]
        return carry
    lax.fori_loop(0, RPW // 16, smem_fill, 0)

    def m0g_body(k, carry):
        idxs = idx_all_v[pl.ds(row0 + k * 16, 16)]
        gstage_v[pl.ds(k * 16, 16)] = plsc.load_gather(m0_v, [idxs])
        return carry
    lax.fori_loop(0, RPW // 16, m0g_body, 0)
    pltpu.sync_copy(gstage_v, m0g_hbm.at[pl.ds(row0, RPW)])

    def fire(r, buf, sem):
        i = idx_my_s[r]
        pltpu.async_copy(j_hbm.at[i], buf, sem)

    def wait_row(buf, sem):
        pltpu.make_async_copy(j_hbm.at[0], buf, sem).wait()

    def extract(buf, stage):
        def col_body(k, carry):
            idxs = idx_all_v[pl.ds(k * 16, 16)]
            stage[pl.ds(k * 16, 16)] = plsc.load_gather(buf, [idxs])
            return carry
        lax.fori_loop(0, T // 16, col_body, 0, unroll=8)

    for q in range(NBUF):
        fire(q, rbufs[q], dsems[q])

    def group_body(j, carry):
        for q in range(NBUF):
            r = NBUF * j + q
            wait_row(rbufs[q], dsems[q])

            @pl.when(j > 0)
            def _():
                pltpu.make_async_copy(stages[q], s_hbm.at[0],
                                      wsems[q]).wait()
            extract(rbufs[q], stages[q])
            pltpu.async_copy(stages[q], s_hbm.at[row0 + r], wsems[q])
            rn = jnp.minimum(NBUF * j + q + NBUF, RPW - 1)
            fire(rn, rbufs[q], dsems[q])
        return carry

    lax.fori_loop(0, RPW // NBUF, group_body, 0)
    for q in range(NBUF):
        wait_row(rbufs[q], dsems[q])
        pltpu.make_async_copy(stages[q], s_hbm.at[0], wsems[q]).wait()


def _sc_extract(J, i_seq, m0):
    mesh = plsc.VectorSubcoreMesh(core_axis_name="c", subcore_axis_name="s")
    f = pl.kernel(
        _sc_extract_body,
        out_type=(
            jax.ShapeDtypeStruct((T, T), jnp.float32),
            jax.ShapeDtypeStruct((T,), jnp.float32),
        ),
        mesh=mesh,
        compiler_params=pltpu.CompilerParams(needs_layout_passes=False),
        scratch_types=[
            pltpu.VMEM((T,), jnp.int32),
            pltpu.VMEM((N,), jnp.float32),
            pltpu.VMEM((N,), jnp.float32),
            pltpu.VMEM((N,), jnp.float32),
            pltpu.VMEM((N,), jnp.float32),
            pltpu.VMEM((N,), jnp.float32),
            pltpu.VMEM((T,), jnp.float32),
            pltpu.VMEM((T,), jnp.float32),
            pltpu.VMEM((T,), jnp.float32),
            pltpu.VMEM((T,), jnp.float32),
            pltpu.VMEM((RPW,), jnp.float32),
            pltpu.SMEM((RPW,), jnp.int32),
            pltpu.SemaphoreType.DMA,
            pltpu.SemaphoreType.DMA,
            pltpu.SemaphoreType.DMA,
            pltpu.SemaphoreType.DMA,
            pltpu.SemaphoreType.DMA,
            pltpu.SemaphoreType.DMA,
            pltpu.SemaphoreType.DMA,
            pltpu.SemaphoreType.DMA,
        ],
    )
    return f(J, i_seq, m0)


def _chunk_body(i_sp, r_sp, prev_sp, m0g_sp, i0_sp,
                srow_ref, sdiag_ref, vals_ref, field_ref):
    c = pl.program_id(0)

    @pl.when(c == 0)
    def _():
        field_ref[...] = jnp.zeros((NCHUNK, K), jnp.float32)

    lanesK = lax.broadcasted_iota(jnp.int32, (1, K), 1)
    corr0 = field_ref[c, :].reshape(1, K)

    def step(tl, carry):
        deltas, vals, corr = carry
        tg = c * K + tl
        i = i_sp[tg]
        r = r_sp[tg]
        pt = prev_sp[tg]
        I = i0_sp[i] + jnp.sum(jnp.where(lanesK == tl, corr, 0.0))
        p = jnp.tanh(I)
        v = jnp.where(p >= r, jnp.float32(1.0), jnp.float32(-1.0))
        ptc = jnp.maximum(pt, 0)
        prow = vals_ref[ptc // K, :].reshape(1, K)
        mp = jnp.sum(jnp.where(lanesK == ptc % K, prow, 0.0))
        mprev = jnp.where(pt >= 0, mp, m0g_sp[tg])
        d = v - mprev
        srow = sdiag_ref[tl, :].reshape(1, K)
        corr = corr + srow * d
        onehot = lanesK == tl
        deltas = jnp.where(onehot, d, deltas)
        vals = jnp.where(onehot, v, vals)
        vals_ref[c, :] = vals.reshape(K)
        return deltas, vals, corr

    z = jnp.zeros((1, K), jnp.float32)
    deltas, _, _ = lax.fori_loop(0, K, step, (z, z, corr0))

    upd = jnp.dot(deltas, srow_ref[...], precision=lax.Precision.HIGHEST,
                  preferred_element_type=jnp.float32)
    field_ref[...] = field_ref[...] + upd.reshape(NCHUNK, K)


def _chunk_scan(i_seq, r_seq, prev, m0g, I0, S):
    grid_spec = pltpu.PrefetchScalarGridSpec(
        num_scalar_prefetch=5,
        grid=(NCHUNK,),
        in_specs=[
            pl.BlockSpec((K, T), lambda c, *sp: (c, 0)),
            pl.BlockSpec((K, K), lambda c, *sp: (c, c)),
        ],
        out_specs=pl.BlockSpec((NCHUNK, K), lambda c, *sp: (0, 0)),
        scratch_shapes=[
            pltpu.VMEM((NCHUNK, K), jnp.float32),
        ],
    )
    return pl.pallas_call(
        _chunk_body,
        grid_spec=grid_spec,
        out_shape=jax.ShapeDtypeStruct((NCHUNK, K), jnp.float32),
    )(i_seq, r_seq, prev, m0g, I0, S, S)


def _sc_scatter_body(m0_hbm, idx_hbm, vals_hbm, keep_hbm, m_hbm,
                     m_v, idx_v, vals_v, keep_v, sem):
    wid = lax.axis_index("s") * 2 + lax.axis_index("c")

    @pl.when(wid == 0)
    def _():
        pltpu.sync_copy(m0_hbm, m_v)
        pltpu.sync_copy(idx_hbm, idx_v)
        pltpu.sync_copy(vals_hbm, vals_v)
        pltpu.sync_copy(keep_hbm, keep_v)

        def body(k, carry):
            sl = pl.ds(k * 16, 16)
            mask = keep_v[sl] != 0
            plsc.store_scatter(m_v, [idx_v[sl]], vals_v[sl], mask=mask)
            return carry
        lax.fori_loop(0, T // 16, body, 0)
        pltpu.sync_copy(m_v, m_hbm)


def _sc_scatter(m0, i_seq, vals, keep):
    mesh = plsc.VectorSubcoreMesh(core_axis_name="c", subcore_axis_name="s")
    f = pl.kernel(
        _sc_scatter_body,
        out_type=jax.ShapeDtypeStruct((N,), jnp.float32),
        mesh=mesh,
        compiler_params=pltpu.CompilerParams(needs_layout_passes=False),
        scratch_types=[
            pltpu.VMEM((N,), jnp.float32),
            pltpu.VMEM((T,), jnp.int32),
            pltpu.VMEM((T,), jnp.float32),
            pltpu.VMEM((T,), jnp.int32),
            pltpu.SemaphoreType.DMA,
        ],
    )
    return f(m0, i_seq, vals, keep)


def kernel(J, h, m0, i_seq, r_seq, steps):
    del steps
    I0 = _matvec(J, m0, h).reshape(N)
    S, m0g = _sc_extract(J, i_seq, m0)

    order = jnp.argsort(i_seq, stable=True)
    io = i_seq[order]
    same_as_prev = jnp.concatenate(
        [jnp.zeros((1,), jnp.bool_), io[1:] == io[:-1]])
    prev_o = jnp.where(same_as_prev,
                       jnp.concatenate([jnp.full((1,), -1, order.dtype),
                                        order[:-1]]),
                       -1).astype(jnp.int32)
    prev = jnp.zeros((T,), jnp.int32).at[order].set(prev_o)
    last_o = jnp.concatenate([io[1:] != io[:-1], jnp.ones((1,), jnp.bool_)])
    keep = jnp.zeros((T,), jnp.int32).at[order].set(last_o.astype(jnp.int32))

    vals = _chunk_scan(i_seq, r_seq, prev, m0g, I0, S)
    return _sc_scatter(m0, i_seq, vals.reshape(T), keep)


# speedup vs baseline: 44.2361x; 1.1412x over previous
import functools

import jax
import jax.numpy as jnp
from jax import lax
from jax.experimental import pallas as pl
from jax.experimental.pallas import tpu as pltpu
from jax.experimental.pallas import tpu_sc as plsc

N = 8192
T = 4096
K = 512
NCHUNK = T // K
RA = 128

NWORK = 32
RPW = T // NWORK
NBUF = 4


def _matvec_body(j_ref, m0_ref, h_ref, out_ref):
    prod = j_ref[...] * m0_ref[...]
    out_ref[...] = jnp.sum(prod, axis=1, keepdims=True) + h_ref[...]


def _matvec(J, m0, h):
    return pl.pallas_call(
        _matvec_body,
        grid=(N // RA,),
        in_specs=[
            pl.BlockSpec((RA, N), lambda b: (b, 0)),
            pl.BlockSpec((1, N), lambda b: (0, 0)),
            pl.BlockSpec((RA, 1), lambda b: (b, 0)),
        ],
        out_specs=pl.BlockSpec((RA, 1), lambda b: (b, 0)),
        out_shape=jax.ShapeDtypeStruct((N, 1), jnp.float32),
    )(J, m0.reshape(1, N), h.reshape(N, 1))


def _sc_extract_body(j_hbm, idx_hbm, m0_hbm, s_hbm, m0g_hbm,
                     idx_all_v, m0_v, rb0, rb1, rb2, rb3,
                     st0, st1, st2, st3, gstage_v, idx_my_s,
                     d0, d1, d2, d3, w0, w1, w2, w3):
    wid = lax.axis_index("s") * 2 + lax.axis_index("c")
    row0 = wid * RPW
    rbufs = (rb0, rb1, rb2, rb3)
    stages = (st0, st1, st2, st3)
    dsems = (d0, d1, d2, d3)
    wsems = (w0, w1, w2, w3)

    pltpu.sync_copy(idx_hbm, idx_all_v)
    pltpu.sync_copy(m0_hbm, m0_v)

    def smem_fill(k, carry):
        vec = idx_all_v[pl.ds(row0 + k * 16, 16)]
        for l in range(16):
            idx_my_s[k * 16 + l] = vec[l]
        return carry
    lax.fori_loop(0, RPW // 16, smem_fill, 0)

    def m0g_body(k, carry):
        idxs = idx_all_v[pl.ds(row0 + k * 16, 16)]
        gstage_v[pl.ds(k * 16, 16)] = plsc.load_gather(m0_v, [idxs])
        return carry
    lax.fori_loop(0, RPW // 16, m0g_body, 0)
    pltpu.sync_copy(gstage_v, m0g_hbm.at[pl.ds(row0, RPW)])

    def fire(r, buf, sem):
        i = idx_my_s[r]
        pltpu.async_copy(j_hbm.at[i], buf, sem)

    def wait_row(buf, sem):
        pltpu.make_async_copy(j_hbm.at[0], buf, sem).wait()

    def extract(buf, stage):
        def col_body(k, carry):
            idxs = idx_all_v[pl.ds(k * 16, 16)]
            stage[pl.ds(k * 16, 16)] = plsc.load_gather(buf, [idxs])
            return carry
        lax.fori_loop(0, T // 16, col_body, 0, unroll=8)

    for q in range(NBUF):
        fire(q, rbufs[q], dsems[q])

    def group_body(j, carry):
        for q in range(NBUF):
            r = NBUF * j + q
            wait_row(rbufs[q], dsems[q])

            @pl.when(j > 0)
            def _():
                pltpu.make_async_copy(stages[q], s_hbm.at[0],
                                      wsems[q]).wait()
            extract(rbufs[q], stages[q])
            pltpu.async_copy(stages[q], s_hbm.at[row0 + r], wsems[q])
            rn = jnp.minimum(NBUF * j + q + NBUF, RPW - 1)
            fire(rn, rbufs[q], dsems[q])
        return carry

    lax.fori_loop(0, RPW // NBUF, group_body, 0)
    for q in range(NBUF):
        wait_row(rbufs[q], dsems[q])
        pltpu.make_async_copy(stages[q], s_hbm.at[0], wsems[q]).wait()


def _sc_extract(J, i_seq, m0):
    mesh = plsc.VectorSubcoreMesh(core_axis_name="c", subcore_axis_name="s")
    f = pl.kernel(
        _sc_extract_body,
        out_type=(
            jax.ShapeDtypeStruct((T, T), jnp.float32),
            jax.ShapeDtypeStruct((T,), jnp.float32),
        ),
        mesh=mesh,
        compiler_params=pltpu.CompilerParams(needs_layout_passes=False),
        scratch_types=[
            pltpu.VMEM((T,), jnp.int32),
            pltpu.VMEM((N,), jnp.float32),
            pltpu.VMEM((N,), jnp.float32),
            pltpu.VMEM((N,), jnp.float32),
            pltpu.VMEM((N,), jnp.float32),
            pltpu.VMEM((N,), jnp.float32),
            pltpu.VMEM((T,), jnp.float32),
            pltpu.VMEM((T,), jnp.float32),
            pltpu.VMEM((T,), jnp.float32),
            pltpu.VMEM((T,), jnp.float32),
            pltpu.VMEM((RPW,), jnp.float32),
            pltpu.SMEM((RPW,), jnp.int32),
            pltpu.SemaphoreType.DMA,
            pltpu.SemaphoreType.DMA,
            pltpu.SemaphoreType.DMA,
            pltpu.SemaphoreType.DMA,
            pltpu.SemaphoreType.DMA,
            pltpu.SemaphoreType.DMA,
            pltpu.SemaphoreType.DMA,
            pltpu.SemaphoreType.DMA,
        ],
    )
    return f(J, i_seq, m0)


def _chunk_body(i_sp, rt_sp, prev_sp, m0g_sp, i0_sp,
                srow_ref, sdiag_ref, vals_ref, field_ref):
    c = pl.program_id(0)

    @pl.when(c == 0)
    def _():
        field_ref[...] = jnp.zeros((NCHUNK, K), jnp.float32)

    lanesK = lax.broadcasted_iota(jnp.int32, (1, K), 1)
    corr0 = field_ref[c, :].reshape(1, K)

    E0 = jnp.sum(jnp.where(lanesK == 0, corr0, 0.0))
    pt0 = prev_sp[c * K]
    ptc0 = jnp.maximum(pt0, 0)
    row0 = vals_ref[ptc0 // K, :].reshape(1, K)
    cand0 = jnp.sum(jnp.where(lanesK == ptc0 % K, row0, 0.0))
    MP0 = jnp.where(pt0 >= 0, cand0, m0g_sp[c * K])

    def step(tl, carry):
        deltas, vals, corr, E, MP = carry
        tg = c * K + tl
        I = i0_sp[i_sp[tg]] + E
        v = jnp.where(I >= rt_sp[tg], jnp.float32(1.0), jnp.float32(-1.0))
        d = v - MP
        srow = sdiag_ref[tl, :].reshape(1, K)

        tgn = jnp.minimum(tg + 1, T - 1)
        onext = lanesK == (tl + 1)
        Epre = jnp.sum(jnp.where(onext, corr, 0.0))
        Ssc = jnp.sum(jnp.where(onext, srow, 0.0))
        E_next = Epre + Ssc * d
        pt1 = prev_sp[tgn]
        ptc1 = jnp.maximum(pt1, 0)
        rowv = vals_ref[ptc1 // K, :].reshape(1, K)
        selrow = jnp.where(pt1 >= c * K, vals, rowv)
        cand = jnp.sum(jnp.where(lanesK == ptc1 % K, selrow, 0.0))
        MP_next = jnp.where(pt1 == tg, v, cand)
        MP_next = jnp.where(pt1 >= 0, MP_next, m0g_sp[tgn])

        corr = corr + srow * d
        onehot = lanesK == tl
        deltas = jnp.where(onehot, d, deltas)
        vals = jnp.where(onehot, v, vals)
        return deltas, vals, corr, E_next, MP_next

    z = jnp.zeros((1, K), jnp.float32)
    deltas, vals, _, _, _ = lax.fori_loop(0, K, step, (z, z, corr0, E0, MP0))
    vals_ref[c, :] = vals.reshape(K)

    upd = jnp.dot(deltas, srow_ref[...], precision=lax.Precision.HIGHEST,
                  preferred_element_type=jnp.float32)
    field_ref[...] = field_ref[...] + upd.reshape(NCHUNK, K)


def _chunk_scan(i_seq, rt_seq, prev, m0g, I0, S):
    grid_spec = pltpu.PrefetchScalarGridSpec(
        num_scalar_prefetch=5,
        grid=(NCHUNK,),
        in_specs=[
            pl.BlockSpec((K, T), lambda c, *sp: (c, 0)),
            pl.BlockSpec((K, K), lambda c, *sp: (c, c)),
        ],
        out_specs=pl.BlockSpec((NCHUNK, K), lambda c, *sp: (0, 0)),
        scratch_shapes=[
            pltpu.VMEM((NCHUNK, K), jnp.float32),
        ],
    )
    return pl.pallas_call(
        _chunk_body,
        grid_spec=grid_spec,
        out_shape=jax.ShapeDtypeStruct((NCHUNK, K), jnp.float32),
    )(i_seq, rt_seq, prev, m0g, I0, S, S)


def _sc_scatter_body(m0_hbm, idx_hbm, vals_hbm, keep_hbm, m_hbm,
                     m_v, idx_v, vals_v, keep_v, sem):
    wid = lax.axis_index("s") * 2 + lax.axis_index("c")

    @pl.when(wid == 0)
    def _():
        pltpu.sync_copy(m0_hbm, m_v)
        pltpu.sync_copy(idx_hbm, idx_v)
        pltpu.sync_copy(vals_hbm, vals_v)
        pltpu.sync_copy(keep_hbm, keep_v)

        def body(k, carry):
            sl = pl.ds(k * 16, 16)
            mask = keep_v[sl] != 0
            plsc.store_scatter(m_v, [idx_v[sl]], vals_v[sl], mask=mask)
            return carry
        lax.fori_loop(0, T // 16, body, 0)
        pltpu.sync_copy(m_v, m_hbm)


def _sc_scatter(m0, i_seq, vals, keep):
    mesh = plsc.VectorSubcoreMesh(core_axis_name="c", subcore_axis_name="s")
    f = pl.kernel(
        _sc_scatter_body,
        out_type=jax.ShapeDtypeStruct((N,), jnp.float32),
        mesh=mesh,
        compiler_params=pltpu.CompilerParams(needs_layout_passes=False),
        scratch_types=[
            pltpu.VMEM((N,), jnp.float32),
            pltpu.VMEM((T,), jnp.int32),
            pltpu.VMEM((T,), jnp.float32),
            pltpu.VMEM((T,), jnp.int32),
            pltpu.SemaphoreType.DMA,
        ],
    )
    return f(m0, i_seq, vals, keep)


def kernel(J, h, m0, i_seq, r_seq, steps):
    del steps
    I0 = _matvec(J, m0, h).reshape(N)
    S, m0g = _sc_extract(J, i_seq, m0)

    order = jnp.argsort(i_seq, stable=True)
    io = i_seq[order]
    same_as_prev = jnp.concatenate(
        [jnp.zeros((1,), jnp.bool_), io[1:] == io[:-1]])
    prev_o = jnp.where(same_as_prev,
                       jnp.concatenate([jnp.full((1,), -1, order.dtype),
                                        order[:-1]]),
                       -1).astype(jnp.int32)
    prev = jnp.zeros((T,), jnp.int32).at[order].set(prev_o)
    last_o = jnp.concatenate([io[1:] != io[:-1], jnp.ones((1,), jnp.bool_)])
    keep = jnp.zeros((T,), jnp.int32).at[order].set(last_o.astype(jnp.int32))

    vals = _chunk_scan(i_seq, jnp.arctanh(r_seq), prev, m0g, I0, S)
    return _sc_scatter(m0, i_seq, vals.reshape(T), keep)


# speedup vs baseline: 61.2025x; 1.3835x over previous
import functools

import jax
import jax.numpy as jnp
from jax import lax
from jax.experimental import pallas as pl
from jax.experimental.pallas import tpu as pltpu
from jax.experimental.pallas import tpu_sc as plsc

N = 8192
T = 4096
K = 512
NCHUNK = T // K
RA = 128

NWORK = 32
RPW = T // NWORK
NBUF = 4


def _matvec_body(j_ref, m0_ref, h_ref, out_ref):
    prod = j_ref[...] * m0_ref[...]
    out_ref[...] = jnp.sum(prod, axis=1, keepdims=True) + h_ref[...]


def _matvec(J, m0, h):
    return pl.pallas_call(
        _matvec_body,
        grid=(N // RA,),
        in_specs=[
            pl.BlockSpec((RA, N), lambda b: (b, 0)),
            pl.BlockSpec((1, N), lambda b: (0, 0)),
            pl.BlockSpec((RA, 1), lambda b: (b, 0)),
        ],
        out_specs=pl.BlockSpec((RA, 1), lambda b: (b, 0)),
        out_shape=jax.ShapeDtypeStruct((N, 1), jnp.float32),
    )(J, m0.reshape(1, N), h.reshape(N, 1))


def _sc_extract_body(j_hbm, idx_hbm, m0_hbm, s_hbm, m0g_hbm,
                     idx_all_v, m0_v, rb0, rb1, rb2, rb3,
                     st0, st1, st2, st3, gstage_v, idx_my_s,
                     d0, d1, d2, d3, w0, w1, w2, w3):
    wid = lax.axis_index("s") * 2 + lax.axis_index("c")
    row0 = wid * RPW
    rbufs = (rb0, rb1, rb2, rb3)
    stages = (st0, st1, st2, st3)
    dsems = (d0, d1, d2, d3)
    wsems = (w0, w1, w2, w3)

    pltpu.sync_copy(idx_hbm, idx_all_v)
    pltpu.sync_copy(m0_hbm, m0_v)

    def smem_fill(k, carry):
        vec = idx_all_v[pl.ds(row0 + k * 16, 16)]
        for l in range(16):
            idx_my_s[k * 16 + l] = vec[l]
        return carry
    lax.fori_loop(0, RPW // 16, smem_fill, 0)

    def m0g_body(k, carry):
        idxs = idx_all_v[pl.ds(row0 + k * 16, 16)]
        gstage_v[pl.ds(k * 16, 16)] = plsc.load_gather(m0_v, [idxs])
        return carry
    lax.fori_loop(0, RPW // 16, m0g_body, 0)
    pltpu.sync_copy(gstage_v, m0g_hbm.at[pl.ds(row0, RPW)])

    def fire(r, buf, sem):
        i = idx_my_s[r]
        pltpu.async_copy(j_hbm.at[i], buf, sem)

    def wait_row(buf, sem):
        pltpu.make_async_copy(j_hbm.at[0], buf, sem).wait()

    def extract(buf, stage):
        def col_body(k, carry):
            idxs = idx_all_v[pl.ds(k * 16, 16)]
            stage[pl.ds(k * 16, 16)] = plsc.load_gather(buf, [idxs])
            return carry
        lax.fori_loop(0, T // 16, col_body, 0, unroll=8)

    for q in range(NBUF):
        fire(q, rbufs[q], dsems[q])

    def group_body(j, carry):
        for q in range(NBUF):
            r = NBUF * j + q
            wait_row(rbufs[q], dsems[q])

            @pl.when(j > 0)
            def _():
                pltpu.make_async_copy(stages[q], s_hbm.at[0],
                                      wsems[q]).wait()
            extract(rbufs[q], stages[q])
            pltpu.async_copy(stages[q], s_hbm.at[row0 + r], wsems[q])
            rn = jnp.minimum(NBUF * j + q + NBUF, RPW - 1)
            fire(rn, rbufs[q], dsems[q])
        return carry

    lax.fori_loop(0, RPW // NBUF, group_body, 0)
    for q in range(NBUF):
        wait_row(rbufs[q], dsems[q])
        pltpu.make_async_copy(stages[q], s_hbm.at[0], wsems[q]).wait()


def _sc_extract(J, i_seq, m0):
    mesh = plsc.VectorSubcoreMesh(core_axis_name="c", subcore_axis_name="s")
    f = pl.kernel(
        _sc_extract_body,
        out_type=(
            jax.ShapeDtypeStruct((T, T), jnp.float32),
            jax.ShapeDtypeStruct((T,), jnp.float32),
        ),
        mesh=mesh,
        compiler_params=pltpu.CompilerParams(needs_layout_passes=False),
        scratch_types=[
            pltpu.VMEM((T,), jnp.int32),
            pltpu.VMEM((N,), jnp.float32),
            pltpu.VMEM((N,), jnp.float32),
            pltpu.VMEM((N,), jnp.float32),
            pltpu.VMEM((N,), jnp.float32),
            pltpu.VMEM((N,), jnp.float32),
            pltpu.VMEM((T,), jnp.float32),
            pltpu.VMEM((T,), jnp.float32),
            pltpu.VMEM((T,), jnp.float32),
            pltpu.VMEM((T,), jnp.float32),
            pltpu.VMEM((RPW,), jnp.float32),
            pltpu.SMEM((RPW,), jnp.int32),
            pltpu.SemaphoreType.DMA,
            pltpu.SemaphoreType.DMA,
            pltpu.SemaphoreType.DMA,
            pltpu.SemaphoreType.DMA,
            pltpu.SemaphoreType.DMA,
            pltpu.SemaphoreType.DMA,
            pltpu.SemaphoreType.DMA,
            pltpu.SemaphoreType.DMA,
        ],
    )
    return f(J, i_seq, m0)


def _chunk_body(i_sp, rt_sp, prev_sp, m0g_sp, i0_sp,
                srow_ref, sdiag_ref, vals_ref, field_ref):
    c = pl.program_id(0)

    @pl.when(c == 0)
    def _():
        field_ref[...] = jnp.zeros((NCHUNK, K), jnp.float32)

    lanesK = lax.broadcasted_iota(jnp.int32, (1, K), 1)
    corr0 = field_ref[c, :].reshape(1, K)

    E0 = jnp.sum(jnp.where(lanesK == 0, corr0, 0.0))
    pt0 = prev_sp[c * K]
    ptc0 = jnp.maximum(pt0, 0)
    row0 = vals_ref[ptc0 // K, :].reshape(1, K)
    cand0 = jnp.sum(jnp.where(lanesK == ptc0 % K, row0, 0.0))
    MP0 = jnp.where(pt0 >= 0, cand0, m0g_sp[c * K])

    def step(tl, carry):
        deltas, vals, corr, E, MP = carry
        tg = c * K + tl
        I = i0_sp[i_sp[tg]] + E
        v = jnp.where(I >= rt_sp[tg], jnp.float32(1.0), jnp.float32(-1.0))
        d = v - MP
        srow = sdiag_ref[tl, :].reshape(1, K)

        tgn = jnp.minimum(tg + 1, T - 1)
        onext = lanesK == (tl + 1)
        Epre = jnp.sum(jnp.where(onext, corr, 0.0))
        Ssc = jnp.sum(jnp.where(onext, srow, 0.0))
        E_next = Epre + Ssc * d
        pt1 = prev_sp[tgn]
        ptc1 = jnp.maximum(pt1, 0)
        rowv = vals_ref[ptc1 // K, :].reshape(1, K)
        selrow = jnp.where(pt1 >= c * K, vals, rowv)
        cand = jnp.sum(jnp.where(lanesK == ptc1 % K, selrow, 0.0))
        MP_next = jnp.where(pt1 == tg, v, cand)
        MP_next = jnp.where(pt1 >= 0, MP_next, m0g_sp[tgn])

        corr = corr + srow * d
        onehot = lanesK == tl
        deltas = jnp.where(onehot, d, deltas)
        vals = jnp.where(onehot, v, vals)
        return deltas, vals, corr, E_next, MP_next

    z = jnp.zeros((1, K), jnp.float32)
    deltas, vals, _, _, _ = lax.fori_loop(0, K, step, (z, z, corr0, E0, MP0),
                                          unroll=2)
    vals_ref[c, :] = vals.reshape(K)

    upd = jnp.dot(deltas, srow_ref[...], precision=lax.Precision.HIGHEST,
                  preferred_element_type=jnp.float32)
    field_ref[...] = field_ref[...] + upd.reshape(NCHUNK, K)


def _chunk_scan(i_seq, rt_seq, prev, m0g, I0, S):
    grid_spec = pltpu.PrefetchScalarGridSpec(
        num_scalar_prefetch=5,
        grid=(NCHUNK,),
        in_specs=[
            pl.BlockSpec((K, T), lambda c, *sp: (c, 0)),
            pl.BlockSpec((K, K), lambda c, *sp: (c, c)),
        ],
        out_specs=pl.BlockSpec((NCHUNK, K), lambda c, *sp: (0, 0)),
        scratch_shapes=[
            pltpu.VMEM((NCHUNK, K), jnp.float32),
        ],
    )
    return pl.pallas_call(
        _chunk_body,
        grid_spec=grid_spec,
        out_shape=jax.ShapeDtypeStruct((NCHUNK, K), jnp.float32),
    )(i_seq, rt_seq, prev, m0g, I0, S, S)


def _sc_scatter_body(m0_hbm, idx_hbm, vals_hbm, keep_hbm, m_hbm,
                     m_v, idx_v, vals_v, keep_v, sem):
    wid = lax.axis_index("s") * 2 + lax.axis_index("c")

    @pl.when(wid == 0)
    def _():
        pltpu.sync_copy(m0_hbm, m_v)
        pltpu.sync_copy(idx_hbm, idx_v)
        pltpu.sync_copy(vals_hbm, vals_v)
        pltpu.sync_copy(keep_hbm, keep_v)

        def body(k, carry):
            sl = pl.ds(k * 16, 16)
            mask = keep_v[sl] != 0
            plsc.store_scatter(m_v, [idx_v[sl]], vals_v[sl], mask=mask)
            return carry
        lax.fori_loop(0, T // 16, body, 0)
        pltpu.sync_copy(m_v, m_hbm)


def _sc_scatter(m0, i_seq, vals, keep):
    mesh = plsc.VectorSubcoreMesh(core_axis_name="c", subcore_axis_name="s")
    f = pl.kernel(
        _sc_scatter_body,
        out_type=jax.ShapeDtypeStruct((N,), jnp.float32),
        mesh=mesh,
        compiler_params=pltpu.CompilerParams(needs_layout_passes=False),
        scratch_types=[
            pltpu.VMEM((N,), jnp.float32),
            pltpu.VMEM((T,), jnp.int32),
            pltpu.VMEM((T,), jnp.float32),
            pltpu.VMEM((T,), jnp.int32),
            pltpu.SemaphoreType.DMA,
        ],
    )
    return f(m0, i_seq, vals, keep)


def kernel(J, h, m0, i_seq, r_seq, steps):
    del steps
    I0 = _matvec(J, m0, h).reshape(N)
    S, m0g = _sc_extract(J, i_seq, m0)

    order = jnp.argsort(i_seq, stable=True)
    io = i_seq[order]
    same_as_prev = jnp.concatenate(
        [jnp.zeros((1,), jnp.bool_), io[1:] == io[:-1]])
    prev_o = jnp.where(same_as_prev,
                       jnp.concatenate([jnp.full((1,), -1, order.dtype),
                                        order[:-1]]),
                       -1).astype(jnp.int32)
    prev = jnp.zeros((T,), jnp.int32).at[order].set(prev_o)
    last_o = jnp.concatenate([io[1:] != io[:-1], jnp.ones((1,), jnp.bool_)])
    keep = jnp.zeros((T,), jnp.int32).at[order].set(last_o.astype(jnp.int32))

    vals = _chunk_scan(i_seq, jnp.arctanh(r_seq), prev, m0g, I0, S)
    return _sc_scatter(m0, i_seq, vals.reshape(T), keep)
